# baseline (speedup 1.0000x reference)
import jax
import jax.numpy as jnp
from jax import lax
from jax.experimental import pallas as pl
from jax.experimental.pallas import tpu as pltpu
from jax.experimental.pallas import tpu_sc as plsc

NUM_POSITIONS = 8192
EMBED_DIM = 1024
B_TOTAL = 4 * 8192

_info = plsc.get_sparse_core_info()
_NC, _NS = _info.num_cores, _info.num_subcores
_NW = _NC * _NS
_B_PER_W = B_TOTAL // _NW
_CHUNK = 32
_N_CHUNKS = _B_PER_W // _CHUNK


def _gather_kernel(x_hbm, w_hbm, out_hbm, idx_v,
                   rows0, rows1, rows2, gs0, gs1, gs2, ws0, ws1, ws2):
    wid = lax.axis_index("s") * _NC + lax.axis_index("c")
    base = wid * _B_PER_W
    pltpu.sync_copy(x_hbm.at[pl.ds(base, _B_PER_W)], idx_v)

    bufs = (rows0, rows1, rows2)
    gsems = (gs0, gs1, gs2)
    wsems = (ws0, ws1, ws2)

    def g_start(i, b):
        pltpu.async_copy(w_hbm.at[idx_v.at[pl.ds(i * _CHUNK, _CHUNK)]],
                         bufs[b], gsems[b])

    def g_wait(b):
        pltpu.make_async_copy(w_hbm.at[pl.ds(0, _CHUNK)], bufs[b],
                              gsems[b]).wait()

    def w_start(i, b):
        pltpu.async_copy(bufs[b], out_hbm.at[pl.ds(base + i * _CHUNK, _CHUNK)],
                         wsems[b])

    def w_wait(b):
        pltpu.make_async_copy(bufs[b], out_hbm.at[pl.ds(base, _CHUNK)],
                              wsems[b]).wait()

    def step(i, b, *, first=False, last=False):
        g_wait(b)
        w_start(i, b)
        if not first:
            w_wait((b + 1) % 3)
        if not last:
            g_start(i + 1, (b + 1) % 3)

    g_start(0, 0)
    step(0, 0, first=True)
    step(1, 1, first=True)

    def body(j, _):
        for s in range(3):
            i = 2 + 3 * j + s

            def one(i=i, b=(2 + s) % 3):
                g_wait(b)
                w_start(i, b)
                w_wait((b + 1) % 3)
                g_start(i + 1, (b + 1) % 3)

            one()
        return ()

    lax.fori_loop(0, 9, body, (), unroll=False)

    step(_N_CHUNKS - 3, (_N_CHUNKS - 3) % 3)
    step(_N_CHUNKS - 2, (_N_CHUNKS - 2) % 3)
    step(_N_CHUNKS - 1, (_N_CHUNKS - 1) % 3, last=True)
    w_wait((_N_CHUNKS - 2) % 3)
    w_wait((_N_CHUNKS - 1) % 3)


@jax.jit
def kernel(x, weight):
    x_flat = x.reshape(B_TOTAL).astype(jnp.int32)
    mesh = plsc.VectorSubcoreMesh(core_axis_name="c", subcore_axis_name="s")
    out = pl.kernel(
        _gather_kernel,
        mesh=mesh,
        out_type=jax.ShapeDtypeStruct((B_TOTAL, EMBED_DIM), jnp.float32),
        scratch_types=[
            pltpu.VMEM((_B_PER_W,), jnp.int32),
            pltpu.VMEM((_CHUNK, EMBED_DIM), jnp.float32),
            pltpu.VMEM((_CHUNK, EMBED_DIM), jnp.float32),
            pltpu.VMEM((_CHUNK, EMBED_DIM), jnp.float32),
            pltpu.SemaphoreType.DMA,
            pltpu.SemaphoreType.DMA,
            pltpu.SemaphoreType.DMA,
            pltpu.SemaphoreType.DMA,
            pltpu.SemaphoreType.DMA,
            pltpu.SemaphoreType.DMA,
        ],
    )(x_flat, weight)
    return out.reshape(x.shape[0], x.shape[1], EMBED_DIM)


# speedup vs baseline: 1.0261x; 1.0261x over previous
import jax
import jax.numpy as jnp
from jax import lax
from jax.experimental import pallas as pl
from jax.experimental.pallas import tpu as pltpu
from jax.experimental.pallas import tpu_sc as plsc

NUM_POSITIONS = 8192
EMBED_DIM = 1024
B_TOTAL = 4 * 8192

_info = plsc.get_sparse_core_info()
_NC, _NS = _info.num_cores, _info.num_subcores
_NW = _NC * _NS
_B_PER_W = B_TOTAL // _NW
_CHUNK = 32
_N_CHUNKS = _B_PER_W // _CHUNK


def _gather_kernel(x_hbm, w_hbm, out_hbm, idx_v,
                   rows0, rows1, rows2, gs0, gs1, gs2, ws0, ws1, ws2):
    wid = lax.axis_index("s") * _NC + lax.axis_index("c")
    base = wid * _B_PER_W
    pltpu.sync_copy(x_hbm.at[pl.ds(base, _B_PER_W)], idx_v)

    bufs = (rows0, rows1, rows2)
    gsems = (gs0, gs1, gs2)
    wsems = (ws0, ws1, ws2)

    def g_start(i, b):
        pltpu.async_copy(w_hbm.at[idx_v.at[pl.ds(i * _CHUNK, _CHUNK)]],
                         bufs[b], gsems[b])

    def g_wait(b):
        pltpu.make_async_copy(w_hbm.at[pl.ds(0, _CHUNK)], bufs[b],
                              gsems[b]).wait()

    def w_start(i, b):
        pltpu.async_copy(bufs[b], out_hbm.at[pl.ds(base + i * _CHUNK, _CHUNK)],
                         wsems[b])

    def w_wait(b):
        pltpu.make_async_copy(bufs[b], out_hbm.at[pl.ds(base, _CHUNK)],
                              wsems[b]).wait()

    g_start(0, 0)
    g_start(1, 1)
    g_wait(0)
    w_start(0, 0)
    g_start(2, 2)
    g_wait(1)
    w_start(1, 1)
    w_wait(0)
    g_start(3, 0)

    def body(j, _):
        for s in range(3):
            i = 2 + 3 * j + s
            b = (2 + s) % 3
            g_wait(b)
            w_start(i, b)
            w_wait((b + 2) % 3)
            g_start(i + 2, (b + 2) % 3)
        return ()

    lax.fori_loop(0, 9, body, (), unroll=False)

    g_wait(2)
    w_start(_N_CHUNKS - 3, 2)
    w_wait(1)
    g_start(_N_CHUNKS - 1, 1)
    g_wait(0)
    w_start(_N_CHUNKS - 2, 0)
    w_wait(2)
    g_wait(1)
    w_start(_N_CHUNKS - 1, 1)
    w_wait(0)
    w_wait(1)


@jax.jit
def kernel(x, weight):
    x_flat = x.reshape(B_TOTAL).astype(jnp.int32)
    mesh = plsc.VectorSubcoreMesh(core_axis_name="c", subcore_axis_name="s")
    out = pl.kernel(
        _gather_kernel,
        mesh=mesh,
        out_type=jax.ShapeDtypeStruct((B_TOTAL, EMBED_DIM), jnp.float32),
        scratch_types=[
            pltpu.VMEM((_B_PER_W,), jnp.int32),
            pltpu.VMEM((_CHUNK, EMBED_DIM), jnp.float32),
            pltpu.VMEM((_CHUNK, EMBED_DIM), jnp.float32),
            pltpu.VMEM((_CHUNK, EMBED_DIM), jnp.float32),
            pltpu.SemaphoreType.DMA,
            pltpu.SemaphoreType.DMA,
            pltpu.SemaphoreType.DMA,
            pltpu.SemaphoreType.DMA,
            pltpu.SemaphoreType.DMA,
            pltpu.SemaphoreType.DMA,
        ],
    )(x_flat, weight)
    return out.reshape(x.shape[0], x.shape[1], EMBED_DIM)


# speedup vs baseline: 1.0345x; 1.0081x over previous
import jax
import jax.numpy as jnp
from jax import lax
from jax.experimental import pallas as pl
from jax.experimental.pallas import tpu as pltpu
from jax.experimental.pallas import tpu_sc as plsc

NUM_POSITIONS = 8192
EMBED_DIM = 1024
B_TOTAL = 4 * 8192

_info = plsc.get_sparse_core_info()
_NC, _NS = _info.num_cores, _info.num_subcores
_NW = _NC * _NS
_B_PER_W = B_TOTAL // _NW
_CHUNK = 16
_N_CHUNKS = _B_PER_W // _CHUNK
_NBUF = 6
_DEPTH = 3


def _gather_kernel(x_hbm, w_hbm, out_hbm, idx_v, bufs_v, gsems, wsems):
    wid = lax.axis_index("s") * _NC + lax.axis_index("c")
    base = wid * _B_PER_W
    pltpu.sync_copy(x_hbm.at[pl.ds(base, _B_PER_W)], idx_v)

    def g_start(i, b):
        pltpu.async_copy(w_hbm.at[idx_v.at[pl.ds(i * _CHUNK, _CHUNK)]],
                         bufs_v.at[b], gsems.at[b])

    def g_wait(b):
        pltpu.make_async_copy(w_hbm.at[pl.ds(0, _CHUNK)], bufs_v.at[b],
                              gsems.at[b]).wait()

    def w_start(i, b):
        pltpu.async_copy(bufs_v.at[b],
                         out_hbm.at[pl.ds(base + i * _CHUNK, _CHUNK)],
                         wsems.at[b])

    def w_wait(b):
        pltpu.make_async_copy(bufs_v.at[b], out_hbm.at[pl.ds(base, _CHUNK)],
                              wsems.at[b]).wait()

    def step(i, *, first=False, last=False):
        b = i % _NBUF
        g_wait(b)
        w_start(i, b)
        if not first:
            w_wait((b + _DEPTH) % _NBUF)
        if not last:
            g_start(i + _DEPTH, (b + _DEPTH) % _NBUF)

    for i in range(_DEPTH):
        g_start(i, i)
    for i in range(_DEPTH):
        step(i, first=True)

    def body(j, _):
        for s in range(_NBUF):
            i = _DEPTH + _NBUF * j + s
            b = (_DEPTH + s) % _NBUF
            g_wait(b)
            w_start(i, b)
            w_wait((b + _DEPTH) % _NBUF)
            g_start(i + _DEPTH, (b + _DEPTH) % _NBUF)
        return ()

    _n_main = (_N_CHUNKS - 2 * _DEPTH - 1) // _NBUF
    lax.fori_loop(0, _n_main, body, (), unroll=False)

    for i in range(_DEPTH + _NBUF * _n_main, _N_CHUNKS):
        step(i, last=(i + _DEPTH >= _N_CHUNKS))
    for i in range(_N_CHUNKS - _DEPTH, _N_CHUNKS):
        w_wait(i % _NBUF)


@jax.jit
def kernel(x, weight):
    x_flat = x.reshape(B_TOTAL).astype(jnp.int32)
    mesh = plsc.VectorSubcoreMesh(core_axis_name="c", subcore_axis_name="s")
    out = pl.kernel(
        _gather_kernel,
        mesh=mesh,
        out_type=jax.ShapeDtypeStruct((B_TOTAL, EMBED_DIM), jnp.float32),
        scratch_types=[
            pltpu.VMEM((_B_PER_W,), jnp.int32),
            pltpu.VMEM((_NBUF, _CHUNK, EMBED_DIM), jnp.float32),
            pltpu.SemaphoreType.DMA((_NBUF,)),
            pltpu.SemaphoreType.DMA((_NBUF,)),
        ],
    )(x_flat, weight)
    return out.reshape(x.shape[0], x.shape[1], EMBED_DIM)
